# Initial kernel scaffold; baseline (speedup 1.0000x reference)
import jax
import jax.numpy as jnp
from jax.experimental import pallas as pl


def kernel(x, edge_index, batch_index, W_rel0, b_rel0, W_root0, W_rel1, b_rel1, W_root1, W_rel2, b_rel2, W_root2, W_rel3, b_rel3, W_root3, W_rel4, b_rel4, W_root4, W1, b1, W2, b2, W3, b3):
    raise NotImplementedError("write your pallas kernel here")


# speedup vs baseline: 2.7874x; 2.7874x over previous
import functools

import jax
import jax.numpy as jnp
from jax import lax
from jax.experimental import pallas as pl
from jax.experimental.pallas import tpu as pltpu
from jax.experimental.pallas import tpu_sc as plsc

_N = 10000
_E = 320000
_B = 128
_C = 10
_H = 512
_ACC_ROWS = 10240
_K = 80
_RB = 400
_NEG = float("-inf")


def _zero_zbuf(zbuf):
    z = jnp.zeros((16,), jnp.float32)

    @pl.loop(0, _K)
    def _(r):
        @pl.loop(0, 8)
        def _(cc):
            zbuf[r, pl.ds(cc * 16, 16)] = z


def _zero_acc(zbuf, acc, sid):
    @pl.loop(0, 8)
    def _(j):
        pltpu.sync_copy(zbuf, acc.at[pl.ds(sid * 640 + j * _K, _K)])


def _edge_pass(src_hbm, dst_hbm, table, sidx, didx, rows, acc, base, nb):
    @pl.loop(0, nb)
    def _(i):
        off = base + i * _K
        pltpu.sync_copy(src_hbm.at[pl.ds(off, _K)], sidx)
        pltpu.sync_copy(dst_hbm.at[pl.ds(off, _K)], didx)
        pltpu.sync_copy(table.at[sidx], rows)
        pltpu.sync_copy(rows, acc.at[didx], add=True)


def _writeout(acc, out2d, sid):
    rpt = 624
    pltpu.sync_copy(acc.at[pl.ds(sid * rpt, rpt)],
                    out2d.at[pl.ds(sid * rpt, rpt)])

    @pl.when(sid == 15)
    def _():
        pltpu.sync_copy(acc.at[pl.ds(16 * rpt, _N - 16 * rpt)],
                        out2d.at[pl.ds(16 * rpt, _N - 16 * rpt)])


def _seg_sum_layer0(x, src, dst):
    mesh = plsc.VectorSubcoreMesh(core_axis_name="core",
                                  subcore_axis_name="subcore")

    @functools.partial(
        pl.kernel,
        out_type=jax.ShapeDtypeStruct((2, _N, 128), jnp.float32),
        mesh=mesh,
        scratch_types=[
            pltpu.VMEM((_K,), jnp.int32),
            pltpu.VMEM((_K,), jnp.int32),
            pltpu.VMEM((_K, 128), jnp.float32),
            pltpu.VMEM((_K, 128), jnp.float32),
            pltpu.VMEM_SHARED((_ACC_ROWS, 128), jnp.float32),
        ],
    )
    def k(x_hbm, src_hbm, dst_hbm, out_hbm, sidx, didx, rows, zbuf, acc):
        core = lax.axis_index("core")
        sid = lax.axis_index("subcore")
        _zero_zbuf(zbuf)
        _zero_acc(zbuf, acc, sid)
        plsc.subcore_barrier()
        half = _E // 2
        per_tile = half // 16
        for cid in (0, 1):
            @pl.when(core == cid)
            def _(cid=cid):
                base = cid * half + sid * per_tile
                _edge_pass(src_hbm, dst_hbm, x_hbm, sidx, didx, rows, acc,
                           base, per_tile // _K)
        plsc.subcore_barrier()
        for cid in (0, 1):
            @pl.when(core == cid)
            def _(cid=cid):
                _writeout(acc, out_hbm.at[cid], sid)

    return k(x, src, dst)


def _seg_sum_chunks(h4, src, dst):
    mesh = plsc.VectorSubcoreMesh(core_axis_name="core",
                                  subcore_axis_name="subcore")

    @functools.partial(
        pl.kernel,
        out_type=jax.ShapeDtypeStruct((4, _N, 128), jnp.float32),
        mesh=mesh,
        scratch_types=[
            pltpu.VMEM((_K,), jnp.int32),
            pltpu.VMEM((_K,), jnp.int32),
            pltpu.VMEM((_K, 128), jnp.float32),
            pltpu.VMEM((_K, 128), jnp.float32),
            pltpu.VMEM_SHARED((_ACC_ROWS, 128), jnp.float32),
        ],
    )
    def k(h4_hbm, src_hbm, dst_hbm, out_hbm, sidx, didx, rows, zbuf, acc):
        core = lax.axis_index("core")
        sid = lax.axis_index("subcore")
        _zero_zbuf(zbuf)
        per_tile = _E // 16
        for cid in (0, 1):
            @pl.when(core == cid)
            def _(cid=cid):
                for j in (0, 1):
                    c = 2 * cid + j
                    _zero_acc(zbuf, acc, sid)
                    plsc.subcore_barrier()
                    _edge_pass(src_hbm, dst_hbm, h4_hbm.at[c], sidx, didx,
                               rows, acc, sid * per_tile, per_tile // _K)
                    plsc.subcore_barrier()
                    _writeout(acc, out_hbm.at[c], sid)
                    plsc.subcore_barrier()

    return k(h4, src, dst)


def _dense0(p, x, Wrel, brel, Wroot):
    def body(p_ref, x_ref, wr_ref, br_ref, wo_ref, o_ref):
        agg = p_ref[0] + p_ref[1]
        res = jnp.tanh(
            jnp.dot(agg, wr_ref[...], preferred_element_type=jnp.float32)
            + jnp.dot(x_ref[...], wo_ref[...],
                      preferred_element_type=jnp.float32)
            + br_ref[...])
        for c in range(4):
            o_ref[c] = res[:, c * 128:(c + 1) * 128]

    return pl.pallas_call(
        body,
        grid=(_N // _RB,),
        in_specs=[
            pl.BlockSpec((2, _RB, 128), lambda i: (0, i, 0)),
            pl.BlockSpec((_RB, 128), lambda i: (i, 0)),
            pl.BlockSpec((128, _H), lambda i: (0, 0)),
            pl.BlockSpec((1, _H), lambda i: (0, 0)),
            pl.BlockSpec((128, _H), lambda i: (0, 0)),
        ],
        out_specs=pl.BlockSpec((4, _RB, 128), lambda i: (0, i, 0)),
        out_shape=jax.ShapeDtypeStruct((4, _N, 128), jnp.float32),
    )(p, x, Wrel, brel.reshape(1, _H), Wroot)


def _dense(agg4, h4, Wrel, brel, Wroot):
    def body(a_ref, h_ref, wr_ref, br_ref, wo_ref, o_ref):
        agg = jnp.concatenate([a_ref[c] for c in range(4)], axis=1)
        h = jnp.concatenate([h_ref[c] for c in range(4)], axis=1)
        res = jnp.tanh(
            jnp.dot(agg, wr_ref[...], preferred_element_type=jnp.float32)
            + jnp.dot(h, wo_ref[...], preferred_element_type=jnp.float32)
            + br_ref[...])
        for c in range(4):
            o_ref[c] = res[:, c * 128:(c + 1) * 128]

    return pl.pallas_call(
        body,
        grid=(_N // _RB,),
        in_specs=[
            pl.BlockSpec((4, _RB, 128), lambda i: (0, i, 0)),
            pl.BlockSpec((4, _RB, 128), lambda i: (0, i, 0)),
            pl.BlockSpec((_H, _H), lambda i: (0, 0)),
            pl.BlockSpec((1, _H), lambda i: (0, 0)),
            pl.BlockSpec((_H, _H), lambda i: (0, 0)),
        ],
        out_specs=pl.BlockSpec((4, _RB, 128), lambda i: (0, i, 0)),
        out_shape=jax.ShapeDtypeStruct((4, _N, 128), jnp.float32),
    )(agg4, h4, Wrel, brel.reshape(1, _H), Wroot)


def _pool_mlp(h4, bidx3, W1, b1, W2, b2, W3, b3):
    G = _N // _RB

    def body(h_ref, ids_ref, w1_ref, b1_ref, w2_ref, b2_ref, w3_ref, b3_ref,
             o_ref, sum_acc, cnt_acc, max_acc):
        i = pl.program_id(0)

        @pl.when(i == 0)
        def _():
            sum_acc[...] = jnp.zeros((_B, _H), jnp.float32)
            cnt_acc[...] = jnp.zeros((_B, _H), jnp.float32)
            max_acc[...] = jnp.full((_B, _H), _NEG, jnp.float32)

        hb = jnp.concatenate([h_ref[c] for c in range(4)], axis=1)
        ids = ids_ref[0]
        iota_b = lax.broadcasted_iota(jnp.int32, (_RB, _B), 1)
        oh = (ids == iota_b).astype(jnp.float32)
        dn = (((0,), (0,)), ((), ()))
        sum_acc[...] += lax.dot_general(oh, hb, dn,
                                        preferred_element_type=jnp.float32)
        cnt_acc[...] += lax.dot_general(oh, jnp.ones((_RB, _H), jnp.float32),
                                        dn, preferred_element_type=jnp.float32)

        first = ids_ref[0, 0, 0]
        last = ids_ref[0, _RB - 1, 0]
        seg_iota = lax.broadcasted_iota(jnp.int32, (_B, 1), 0)

        def upd(b, _):
            mask = ids == b
            m = jnp.max(jnp.where(mask, hb, _NEG), axis=0,
                        keepdims=True)
            sel = seg_iota == b
            max_acc[...] = jnp.maximum(max_acc[...],
                                       jnp.where(sel, m, _NEG))
            return 0

        lax.fori_loop(first, last + 1, upd, 0)

        @pl.when(i == G - 1)
        def _():
            cnt = cnt_acc[...]
            mean_p = sum_acc[...] / jnp.maximum(cnt, 1.0)
            max_p = jnp.where(cnt > 0.0, max_acc[...], 0.0)
            g = jnp.concatenate([max_p, mean_p], axis=1)
            g = jnp.tanh(jnp.dot(g, w1_ref[...],
                                 preferred_element_type=jnp.float32)
                         + b1_ref[...])
            g = jnp.tanh(jnp.dot(g, w2_ref[...],
                                 preferred_element_type=jnp.float32)
                         + b2_ref[...])
            logits = jnp.dot(g, w3_ref[...],
                             preferred_element_type=jnp.float32) + b3_ref[...]
            mx = jnp.max(logits, axis=1, keepdims=True)
            sh = logits - mx
            lse = jnp.log(jnp.sum(jnp.exp(sh), axis=1, keepdims=True))
            o_ref[...] = sh - lse

    return pl.pallas_call(
        body,
        grid=(G,),
        in_specs=[
            pl.BlockSpec((4, _RB, 128), lambda i: (0, i, 0)),
            pl.BlockSpec((1, _RB, 1), lambda i: (i, 0, 0)),
            pl.BlockSpec((2 * _H, _H), lambda i: (0, 0)),
            pl.BlockSpec((1, _H), lambda i: (0, 0)),
            pl.BlockSpec((_H, _H), lambda i: (0, 0)),
            pl.BlockSpec((1, _H), lambda i: (0, 0)),
            pl.BlockSpec((_H, _C), lambda i: (0, 0)),
            pl.BlockSpec((1, _C), lambda i: (0, 0)),
        ],
        out_specs=pl.BlockSpec((_B, _C), lambda i: (0, 0)),
        out_shape=jax.ShapeDtypeStruct((_B, _C), jnp.float32),
        scratch_shapes=[
            pltpu.VMEM((_B, _H), jnp.float32),
            pltpu.VMEM((_B, _H), jnp.float32),
            pltpu.VMEM((_B, _H), jnp.float32),
        ],
    )(h4, bidx3, W1, b1.reshape(1, _H), W2, b2.reshape(1, _H), W3,
      b3.reshape(1, _C))


def kernel(x, edge_index, batch_index, W_rel0, b_rel0, W_root0,
           W_rel1, b_rel1, W_root1, W_rel2, b_rel2, W_root2,
           W_rel3, b_rel3, W_root3, W_rel4, b_rel4, W_root4,
           W1, b1, W2, b2, W3, b3):
    src = edge_index[0]
    dst = edge_index[1]
    bidx3 = batch_index.reshape(_N // _RB, _RB, 1)

    p = _seg_sum_layer0(x, src, dst)
    h4 = _dense0(p, x, W_rel0, b_rel0, W_root0)
    for Wr, br, Wo in ((W_rel1, b_rel1, W_root1), (W_rel2, b_rel2, W_root2),
                       (W_rel3, b_rel3, W_root3), (W_rel4, b_rel4, W_root4)):
        agg4 = _seg_sum_chunks(h4, src, dst)
        h4 = _dense(agg4, h4, Wr, br, Wo)
    return _pool_mlp(h4, bidx3, W1, b1, W2, b2, W3, b3)


# speedup vs baseline: 6.9187x; 2.4821x over previous
import functools

import jax
import jax.numpy as jnp
from jax import lax
from jax.experimental import pallas as pl
from jax.experimental.pallas import tpu as pltpu
from jax.experimental.pallas import tpu_sc as plsc

_N = 10000
_E = 320000
_B = 128
_C = 10
_H = 512
_ACC_ROWS = 10240
_K = 100
_RB = 400
_NEG = float("-inf")


def _zero_zbuf(zbuf):
    z = jnp.zeros((16,), jnp.float32)

    @pl.loop(0, 80)
    def _(r):
        @pl.loop(0, 8)
        def _(cc):
            zbuf[r, pl.ds(cc * 16, 16)] = z


def _zero_acc(zbuf, acc, sid):
    @pl.loop(0, 8)
    def _(j):
        pltpu.sync_copy(zbuf, acc.at[pl.ds(sid * 640 + j * 80, 80)])


_BB = 20


def _edge_pass(table, sidx_all, didx_all, r0, r1, acc, sem0, sem1, nb):
    def gstart(j, buf, sem):
        pltpu.make_async_copy(table.at[sidx_all.at[j]], buf, sem).start()

    def gwait(j, buf, sem):
        pltpu.make_async_copy(table.at[sidx_all.at[j]], buf, sem).wait()

    def scat(j, buf):
        pltpu.sync_copy(buf, acc.at[didx_all.at[j]], add=True)

    gstart(0, r0, sem0)

    @pl.loop(0, nb - 2, step=2)
    def _(i):
        gstart(i + 1, r1, sem1)
        gwait(i, r0, sem0)
        scat(i, r0)
        gstart(i + 2, r0, sem0)
        gwait(i + 1, r1, sem1)
        scat(i + 1, r1)

    gstart(nb - 1, r1, sem1)
    gwait(nb - 2, r0, sem0)
    scat(nb - 2, r0)
    gwait(nb - 1, r1, sem1)
    scat(nb - 1, r1)


def _edge_blocks(src_tile, dst_tile, table, sidx, didx, r0, r1, acc,
                 sem0, sem1, nblk):
    @pl.loop(0, nblk)
    def _(b):
        pltpu.sync_copy(src_tile.at[b], sidx)
        pltpu.sync_copy(dst_tile.at[b], didx)
        _edge_pass(table, sidx, didx, r0, r1, acc, sem0, sem1, _BB)


def _writeout(acc, out2d, sid):
    rpt = 624
    pltpu.sync_copy(acc.at[pl.ds(sid * rpt, rpt)],
                    out2d.at[pl.ds(sid * rpt, rpt)])

    @pl.when(sid == 15)
    def _():
        pltpu.sync_copy(acc.at[pl.ds(16 * rpt, _N - 16 * rpt)],
                        out2d.at[pl.ds(16 * rpt, _N - 16 * rpt)])


def _seg_sum_layer0(x, src4, dst4):
    nblk = src4.shape[1]
    mesh = plsc.VectorSubcoreMesh(core_axis_name="core",
                                  subcore_axis_name="subcore")

    @functools.partial(
        pl.kernel,
        out_type=jax.ShapeDtypeStruct((2, _N, 128), jnp.float32),
        mesh=mesh,
        scratch_types=[
            pltpu.VMEM((_BB, _K), jnp.int32),
            pltpu.VMEM((_BB, _K), jnp.int32),
            pltpu.VMEM((_K, 128), jnp.float32),
            pltpu.VMEM((_K, 128), jnp.float32),
            pltpu.VMEM((80, 128), jnp.float32),
            pltpu.VMEM_SHARED((_ACC_ROWS, 128), jnp.float32),
            pltpu.SemaphoreType.DMA,
            pltpu.SemaphoreType.DMA,
        ],
    )
    def k(x_hbm, src_hbm, dst_hbm, out_hbm, sidx, didx, r0, r1, zbuf, acc,
          sem0, sem1):
        core = lax.axis_index("core")
        sid = lax.axis_index("subcore")
        tid = core * 16 + sid
        _zero_zbuf(zbuf)
        _zero_acc(zbuf, acc, sid)
        plsc.subcore_barrier()
        _edge_blocks(src_hbm.at[tid], dst_hbm.at[tid], x_hbm, sidx, didx,
                     r0, r1, acc, sem0, sem1, nblk)
        plsc.subcore_barrier()
        for cid in (0, 1):
            @pl.when(core == cid)
            def _(cid=cid):
                _writeout(acc, out_hbm.at[cid], sid)

    return k(x, src4, dst4)


def _seg_sum_chunks(h4, src4, dst4):
    nblk = src4.shape[1]
    mesh = plsc.VectorSubcoreMesh(core_axis_name="core",
                                  subcore_axis_name="subcore")

    @functools.partial(
        pl.kernel,
        out_type=jax.ShapeDtypeStruct((4, _N, 128), jnp.float32),
        mesh=mesh,
        scratch_types=[
            pltpu.VMEM((_BB, _K), jnp.int32),
            pltpu.VMEM((_BB, _K), jnp.int32),
            pltpu.VMEM((_K, 128), jnp.float32),
            pltpu.VMEM((_K, 128), jnp.float32),
            pltpu.VMEM((80, 128), jnp.float32),
            pltpu.VMEM_SHARED((_ACC_ROWS, 128), jnp.float32),
            pltpu.SemaphoreType.DMA,
            pltpu.SemaphoreType.DMA,
        ],
    )
    def k(h4_hbm, src_hbm, dst_hbm, out_hbm, sidx, didx, r0, r1, zbuf, acc,
          sem0, sem1):
        core = lax.axis_index("core")
        sid = lax.axis_index("subcore")
        _zero_zbuf(zbuf)
        for cid in (0, 1):
            @pl.when(core == cid)
            def _(cid=cid):
                for j in (0, 1):
                    c = 2 * cid + j
                    _zero_acc(zbuf, acc, sid)
                    plsc.subcore_barrier()
                    _edge_blocks(src_hbm.at[sid], dst_hbm.at[sid],
                                 h4_hbm.at[c], sidx, didx, r0, r1, acc,
                                 sem0, sem1, nblk)
                    plsc.subcore_barrier()
                    _writeout(acc, out_hbm.at[c], sid)
                    plsc.subcore_barrier()

    return k(h4, src4, dst4)


def _dense0(p, x, Wrel, brel, Wroot):
    def body(p_ref, x_ref, wr_ref, br_ref, wo_ref, o_ref):
        agg = p_ref[0] + p_ref[1]
        res = jnp.tanh(
            jnp.dot(agg, wr_ref[...], preferred_element_type=jnp.float32)
            + jnp.dot(x_ref[...], wo_ref[...],
                      preferred_element_type=jnp.float32)
            + br_ref[...])
        for c in range(4):
            o_ref[c] = res[:, c * 128:(c + 1) * 128]

    return pl.pallas_call(
        body,
        grid=(_N // _RB,),
        in_specs=[
            pl.BlockSpec((2, _RB, 128), lambda i: (0, i, 0)),
            pl.BlockSpec((_RB, 128), lambda i: (i, 0)),
            pl.BlockSpec((128, _H), lambda i: (0, 0)),
            pl.BlockSpec((1, _H), lambda i: (0, 0)),
            pl.BlockSpec((128, _H), lambda i: (0, 0)),
        ],
        out_specs=pl.BlockSpec((4, _RB, 128), lambda i: (0, i, 0)),
        out_shape=jax.ShapeDtypeStruct((4, _N, 128), jnp.float32),
    )(p, x, Wrel, brel.reshape(1, _H), Wroot)


def _dense(agg4, h4, Wrel, brel, Wroot):
    def body(a_ref, h_ref, wr_ref, br_ref, wo_ref, o_ref):
        agg = jnp.concatenate([a_ref[c] for c in range(4)], axis=1)
        h = jnp.concatenate([h_ref[c] for c in range(4)], axis=1)
        res = jnp.tanh(
            jnp.dot(agg, wr_ref[...], preferred_element_type=jnp.float32)
            + jnp.dot(h, wo_ref[...], preferred_element_type=jnp.float32)
            + br_ref[...])
        for c in range(4):
            o_ref[c] = res[:, c * 128:(c + 1) * 128]

    return pl.pallas_call(
        body,
        grid=(_N // _RB,),
        in_specs=[
            pl.BlockSpec((4, _RB, 128), lambda i: (0, i, 0)),
            pl.BlockSpec((4, _RB, 128), lambda i: (0, i, 0)),
            pl.BlockSpec((_H, _H), lambda i: (0, 0)),
            pl.BlockSpec((1, _H), lambda i: (0, 0)),
            pl.BlockSpec((_H, _H), lambda i: (0, 0)),
        ],
        out_specs=pl.BlockSpec((4, _RB, 128), lambda i: (0, i, 0)),
        out_shape=jax.ShapeDtypeStruct((4, _N, 128), jnp.float32),
    )(agg4, h4, Wrel, brel.reshape(1, _H), Wroot)


def _pool_mlp(h4, bidx3, W1, b1, W2, b2, W3, b3):
    G = _N // _RB

    def body(h_ref, ids_ref, w1_ref, b1_ref, w2_ref, b2_ref, w3_ref, b3_ref,
             o_ref, sum_acc, cnt_acc, max_acc):
        i = pl.program_id(0)

        @pl.when(i == 0)
        def _():
            sum_acc[...] = jnp.zeros((_B, _H), jnp.float32)
            cnt_acc[...] = jnp.zeros((_B, _H), jnp.float32)
            max_acc[...] = jnp.full((_B, _H), _NEG, jnp.float32)

        hb = jnp.concatenate([h_ref[c] for c in range(4)], axis=1)
        ids = ids_ref[0]
        iota_b = lax.broadcasted_iota(jnp.int32, (_RB, _B), 1)
        oh = (ids == iota_b).astype(jnp.float32)
        dn = (((0,), (0,)), ((), ()))
        sum_acc[...] += lax.dot_general(oh, hb, dn,
                                        preferred_element_type=jnp.float32)
        cnt_acc[...] += lax.dot_general(oh, jnp.ones((_RB, _H), jnp.float32),
                                        dn, preferred_element_type=jnp.float32)

        first = ids_ref[0, 0, 0]
        last = ids_ref[0, _RB - 1, 0]
        seg_iota = lax.broadcasted_iota(jnp.int32, (_B, 1), 0)

        def upd(b, _):
            mask = ids == b
            m = jnp.max(jnp.where(mask, hb, _NEG), axis=0,
                        keepdims=True)
            sel = seg_iota == b
            max_acc[...] = jnp.maximum(max_acc[...],
                                       jnp.where(sel, m, _NEG))
            return 0

        lax.fori_loop(first, last + 1, upd, 0)

        @pl.when(i == G - 1)
        def _():
            cnt = cnt_acc[...]
            mean_p = sum_acc[...] / jnp.maximum(cnt, 1.0)
            max_p = jnp.where(cnt > 0.0, max_acc[...], 0.0)
            g = jnp.concatenate([max_p, mean_p], axis=1)
            g = jnp.tanh(jnp.dot(g, w1_ref[...],
                                 preferred_element_type=jnp.float32)
                         + b1_ref[...])
            g = jnp.tanh(jnp.dot(g, w2_ref[...],
                                 preferred_element_type=jnp.float32)
                         + b2_ref[...])
            logits = jnp.dot(g, w3_ref[...],
                             preferred_element_type=jnp.float32) + b3_ref[...]
            mx = jnp.max(logits, axis=1, keepdims=True)
            sh = logits - mx
            lse = jnp.log(jnp.sum(jnp.exp(sh), axis=1, keepdims=True))
            o_ref[...] = sh - lse

    return pl.pallas_call(
        body,
        grid=(G,),
        in_specs=[
            pl.BlockSpec((4, _RB, 128), lambda i: (0, i, 0)),
            pl.BlockSpec((1, _RB, 1), lambda i: (i, 0, 0)),
            pl.BlockSpec((2 * _H, _H), lambda i: (0, 0)),
            pl.BlockSpec((1, _H), lambda i: (0, 0)),
            pl.BlockSpec((_H, _H), lambda i: (0, 0)),
            pl.BlockSpec((1, _H), lambda i: (0, 0)),
            pl.BlockSpec((_H, _C), lambda i: (0, 0)),
            pl.BlockSpec((1, _C), lambda i: (0, 0)),
        ],
        out_specs=pl.BlockSpec((_B, _C), lambda i: (0, 0)),
        out_shape=jax.ShapeDtypeStruct((_B, _C), jnp.float32),
        scratch_shapes=[
            pltpu.VMEM((_B, _H), jnp.float32),
            pltpu.VMEM((_B, _H), jnp.float32),
            pltpu.VMEM((_B, _H), jnp.float32),
        ],
    )(h4, bidx3, W1, b1.reshape(1, _H), W2, b2.reshape(1, _H), W3,
      b3.reshape(1, _C))


def kernel(x, edge_index, batch_index, W_rel0, b_rel0, W_root0,
           W_rel1, b_rel1, W_root1, W_rel2, b_rel2, W_root2,
           W_rel3, b_rel3, W_root3, W_rel4, b_rel4, W_root4,
           W1, b1, W2, b2, W3, b3):
    src = edge_index[0]
    dst = edge_index[1]
    src32 = src.reshape(32, _E // 32 // (_BB * _K), _BB, _K)
    dst32 = dst.reshape(32, _E // 32 // (_BB * _K), _BB, _K)
    src16 = src.reshape(16, _E // 16 // (_BB * _K), _BB, _K)
    dst16 = dst.reshape(16, _E // 16 // (_BB * _K), _BB, _K)
    bidx3 = batch_index.reshape(_N // _RB, _RB, 1)

    p = _seg_sum_layer0(x, src32, dst32)
    h4 = _dense0(p, x, W_rel0, b_rel0, W_root0)
    for Wr, br, Wo in ((W_rel1, b_rel1, W_root1), (W_rel2, b_rel2, W_root2),
                       (W_rel3, b_rel3, W_root3), (W_rel4, b_rel4, W_root4)):
        agg4 = _seg_sum_chunks(h4, src16, dst16)
        h4 = _dense(agg4, h4, Wr, br, Wo)
    return _pool_mlp(h4, bidx3, W1, b1, W2, b2, W3, b3)


# speedup vs baseline: 7.2290x; 1.0449x over previous
import functools

import jax
import jax.numpy as jnp
from jax import lax
from jax.experimental import pallas as pl
from jax.experimental.pallas import tpu as pltpu
from jax.experimental.pallas import tpu_sc as plsc

_N = 10000
_E = 320000
_B = 128
_C = 10
_H = 512
_ACC_ROWS = 10240
_K = 100
_RB = 400
_NEG = float("-inf")


def _zero_zbuf(zbuf):
    z = jnp.zeros((16,), jnp.float32)

    @pl.loop(0, 40)
    def _(r):
        @pl.loop(0, 8)
        def _(cc):
            zbuf[r, pl.ds(cc * 16, 16)] = z


def _zero_acc(zbuf, acc, sid):
    @pl.loop(0, 16)
    def _(j):
        pltpu.sync_copy(zbuf, acc.at[pl.ds(sid * 640 + j * 40, 40)])


def _edge_pass(table, sidx_all, didx_all, r0, r1, acc, sem0, sem1, nb):
    def gstart(j, buf, sem):
        pltpu.make_async_copy(table.at[sidx_all.at[j]], buf, sem).start()

    def gwait(j, buf, sem):
        pltpu.make_async_copy(table.at[sidx_all.at[j]], buf, sem).wait()

    def scat(j, buf):
        pltpu.sync_copy(buf, acc.at[didx_all.at[j]], add=True)

    gstart(0, r0, sem0)

    @pl.loop(0, nb - 2, step=2)
    def _(i):
        gstart(i + 1, r1, sem1)
        gwait(i, r0, sem0)
        scat(i, r0)
        gstart(i + 2, r0, sem0)
        gwait(i + 1, r1, sem1)
        scat(i + 1, r1)

    gstart(nb - 1, r1, sem1)
    gwait(nb - 2, r0, sem0)
    scat(nb - 2, r0)
    gwait(nb - 1, r1, sem1)
    scat(nb - 1, r1)


def _edge_blocks(src_tile, dst_tile, table, sidx2, didx2, r0, r1, acc,
                 sem0, sem1, isems, nblk, bb):
    def istart(b, p):
        pltpu.make_async_copy(src_tile.at[b], sidx2.at[p], isems[p]).start()
        pltpu.make_async_copy(dst_tile.at[b], didx2.at[p], isems[p]).start()

    def iwait(b, p):
        pltpu.make_async_copy(src_tile.at[b], sidx2.at[p], isems[p]).wait()
        pltpu.make_async_copy(dst_tile.at[b], didx2.at[p], isems[p]).wait()

    istart(0, 0)

    @pl.loop(0, nblk)
    def _(b):
        for p in (0, 1):
            @pl.when(lax.rem(b, 2) == p)
            def _(p=p):
                iwait(b, p)

                @pl.when(b + 1 < nblk)
                def _():
                    istart(b + 1, 1 - p)

                _edge_pass(table, sidx2.at[p], didx2.at[p], r0, r1, acc,
                           sem0, sem1, bb)


def _writeout(acc, out2d, sid):
    rpt = 624
    pltpu.sync_copy(acc.at[pl.ds(sid * rpt, rpt)],
                    out2d.at[pl.ds(sid * rpt, rpt)])

    @pl.when(sid == 15)
    def _():
        pltpu.sync_copy(acc.at[pl.ds(16 * rpt, _N - 16 * rpt)],
                        out2d.at[pl.ds(16 * rpt, _N - 16 * rpt)])


def _seg_sum_layer0(x, src4, dst4):
    nblk, bb = src4.shape[1], src4.shape[2]
    mesh = plsc.VectorSubcoreMesh(core_axis_name="core",
                                  subcore_axis_name="subcore")

    @functools.partial(
        pl.kernel,
        out_type=jax.ShapeDtypeStruct((2, _N, 128), jnp.float32),
        mesh=mesh,
        scratch_types=[
            pltpu.VMEM((2, bb, _K), jnp.int32),
            pltpu.VMEM((2, bb, _K), jnp.int32),
            pltpu.VMEM((_K, 128), jnp.float32),
            pltpu.VMEM((_K, 128), jnp.float32),
            pltpu.VMEM((40, 128), jnp.float32),
            pltpu.VMEM_SHARED((_ACC_ROWS, 128), jnp.float32),
            pltpu.SemaphoreType.DMA,
            pltpu.SemaphoreType.DMA,
            pltpu.SemaphoreType.DMA,
            pltpu.SemaphoreType.DMA,
        ],
    )
    def k(x_hbm, src_hbm, dst_hbm, out_hbm, sidx2, didx2, r0, r1, zbuf, acc,
          sem0, sem1, isem0, isem1):
        core = lax.axis_index("core")
        sid = lax.axis_index("subcore")
        tid = core * 16 + sid
        _zero_zbuf(zbuf)
        _zero_acc(zbuf, acc, sid)
        plsc.subcore_barrier()
        _edge_blocks(src_hbm.at[tid], dst_hbm.at[tid], x_hbm, sidx2, didx2,
                     r0, r1, acc, sem0, sem1, (isem0, isem1), nblk, bb)
        plsc.subcore_barrier()
        for cid in (0, 1):
            @pl.when(core == cid)
            def _(cid=cid):
                _writeout(acc, out_hbm.at[cid], sid)

    return k(x, src4, dst4)


def _seg_sum_chunks(h4, src4, dst4):
    nblk, bb = src4.shape[1], src4.shape[2]
    mesh = plsc.VectorSubcoreMesh(core_axis_name="core",
                                  subcore_axis_name="subcore")

    @functools.partial(
        pl.kernel,
        out_type=jax.ShapeDtypeStruct((4, _N, 128), jnp.float32),
        mesh=mesh,
        scratch_types=[
            pltpu.VMEM((2, bb, _K), jnp.int32),
            pltpu.VMEM((2, bb, _K), jnp.int32),
            pltpu.VMEM((_K, 128), jnp.float32),
            pltpu.VMEM((_K, 128), jnp.float32),
            pltpu.VMEM((40, 128), jnp.float32),
            pltpu.VMEM_SHARED((_ACC_ROWS, 128), jnp.float32),
            pltpu.SemaphoreType.DMA,
            pltpu.SemaphoreType.DMA,
            pltpu.SemaphoreType.DMA,
            pltpu.SemaphoreType.DMA,
        ],
    )
    def k(h4_hbm, src_hbm, dst_hbm, out_hbm, sidx2, didx2, r0, r1, zbuf, acc,
          sem0, sem1, isem0, isem1):
        core = lax.axis_index("core")
        sid = lax.axis_index("subcore")
        _zero_zbuf(zbuf)
        for cid in (0, 1):
            @pl.when(core == cid)
            def _(cid=cid):
                for j in (0, 1):
                    c = 2 * cid + j
                    _zero_acc(zbuf, acc, sid)
                    plsc.subcore_barrier()
                    _edge_blocks(src_hbm.at[sid], dst_hbm.at[sid],
                                 h4_hbm.at[c], sidx2, didx2, r0, r1, acc,
                                 sem0, sem1, (isem0, isem1), nblk, bb)
                    plsc.subcore_barrier()
                    _writeout(acc, out_hbm.at[c], sid)
                    plsc.subcore_barrier()

    return k(h4, src4, dst4)


def _dense0(p, x, Wrel, brel, Wroot):
    def body(p_ref, x_ref, wr_ref, br_ref, wo_ref, o_ref):
        agg = p_ref[0] + p_ref[1]
        res = jnp.tanh(
            jnp.dot(agg, wr_ref[...], preferred_element_type=jnp.float32)
            + jnp.dot(x_ref[...], wo_ref[...],
                      preferred_element_type=jnp.float32)
            + br_ref[...])
        for c in range(4):
            o_ref[c] = res[:, c * 128:(c + 1) * 128]

    return pl.pallas_call(
        body,
        grid=(_N // _RB,),
        in_specs=[
            pl.BlockSpec((2, _RB, 128), lambda i: (0, i, 0)),
            pl.BlockSpec((_RB, 128), lambda i: (i, 0)),
            pl.BlockSpec((128, _H), lambda i: (0, 0)),
            pl.BlockSpec((1, _H), lambda i: (0, 0)),
            pl.BlockSpec((128, _H), lambda i: (0, 0)),
        ],
        out_specs=pl.BlockSpec((4, _RB, 128), lambda i: (0, i, 0)),
        out_shape=jax.ShapeDtypeStruct((4, _N, 128), jnp.float32),
    )(p, x, Wrel, brel.reshape(1, _H), Wroot)


def _dense(agg4, h4, Wrel, brel, Wroot):
    def body(a_ref, h_ref, wr_ref, br_ref, wo_ref, o_ref):
        agg = jnp.concatenate([a_ref[c] for c in range(4)], axis=1)
        h = jnp.concatenate([h_ref[c] for c in range(4)], axis=1)
        res = jnp.tanh(
            jnp.dot(agg, wr_ref[...], preferred_element_type=jnp.float32)
            + jnp.dot(h, wo_ref[...], preferred_element_type=jnp.float32)
            + br_ref[...])
        for c in range(4):
            o_ref[c] = res[:, c * 128:(c + 1) * 128]

    return pl.pallas_call(
        body,
        grid=(_N // _RB,),
        in_specs=[
            pl.BlockSpec((4, _RB, 128), lambda i: (0, i, 0)),
            pl.BlockSpec((4, _RB, 128), lambda i: (0, i, 0)),
            pl.BlockSpec((_H, _H), lambda i: (0, 0)),
            pl.BlockSpec((1, _H), lambda i: (0, 0)),
            pl.BlockSpec((_H, _H), lambda i: (0, 0)),
        ],
        out_specs=pl.BlockSpec((4, _RB, 128), lambda i: (0, i, 0)),
        out_shape=jax.ShapeDtypeStruct((4, _N, 128), jnp.float32),
    )(agg4, h4, Wrel, brel.reshape(1, _H), Wroot)


def _pool_mlp(h4, bidx3, W1, b1, W2, b2, W3, b3):
    G = _N // _RB

    def body(h_ref, ids_ref, w1_ref, b1_ref, w2_ref, b2_ref, w3_ref, b3_ref,
             o_ref, sum_acc, cnt_acc, max_acc):
        i = pl.program_id(0)

        @pl.when(i == 0)
        def _():
            sum_acc[...] = jnp.zeros((_B, _H), jnp.float32)
            cnt_acc[...] = jnp.zeros((_B, _H), jnp.float32)
            max_acc[...] = jnp.full((_B, _H), _NEG, jnp.float32)

        hb = jnp.concatenate([h_ref[c] for c in range(4)], axis=1)
        ids = ids_ref[0]
        iota_b = lax.broadcasted_iota(jnp.int32, (_RB, _B), 1)
        oh = (ids == iota_b).astype(jnp.float32)
        dn = (((0,), (0,)), ((), ()))
        sum_acc[...] += lax.dot_general(oh, hb, dn,
                                        preferred_element_type=jnp.float32)
        cnt_acc[...] += lax.dot_general(oh, jnp.ones((_RB, _H), jnp.float32),
                                        dn, preferred_element_type=jnp.float32)

        first = ids_ref[0, 0, 0]
        last = ids_ref[0, _RB - 1, 0]
        seg_iota = lax.broadcasted_iota(jnp.int32, (_B, 1), 0)

        def upd(b, _):
            mask = ids == b
            m = jnp.max(jnp.where(mask, hb, _NEG), axis=0,
                        keepdims=True)
            sel = seg_iota == b
            max_acc[...] = jnp.maximum(max_acc[...],
                                       jnp.where(sel, m, _NEG))
            return 0

        lax.fori_loop(first, last + 1, upd, 0)

        @pl.when(i == G - 1)
        def _():
            cnt = cnt_acc[...]
            mean_p = sum_acc[...] / jnp.maximum(cnt, 1.0)
            max_p = jnp.where(cnt > 0.0, max_acc[...], 0.0)
            g = jnp.concatenate([max_p, mean_p], axis=1)
            g = jnp.tanh(jnp.dot(g, w1_ref[...],
                                 preferred_element_type=jnp.float32)
                         + b1_ref[...])
            g = jnp.tanh(jnp.dot(g, w2_ref[...],
                                 preferred_element_type=jnp.float32)
                         + b2_ref[...])
            logits = jnp.dot(g, w3_ref[...],
                             preferred_element_type=jnp.float32) + b3_ref[...]
            mx = jnp.max(logits, axis=1, keepdims=True)
            sh = logits - mx
            lse = jnp.log(jnp.sum(jnp.exp(sh), axis=1, keepdims=True))
            o_ref[...] = sh - lse

    return pl.pallas_call(
        body,
        grid=(G,),
        in_specs=[
            pl.BlockSpec((4, _RB, 128), lambda i: (0, i, 0)),
            pl.BlockSpec((1, _RB, 1), lambda i: (i, 0, 0)),
            pl.BlockSpec((2 * _H, _H), lambda i: (0, 0)),
            pl.BlockSpec((1, _H), lambda i: (0, 0)),
            pl.BlockSpec((_H, _H), lambda i: (0, 0)),
            pl.BlockSpec((1, _H), lambda i: (0, 0)),
            pl.BlockSpec((_H, _C), lambda i: (0, 0)),
            pl.BlockSpec((1, _C), lambda i: (0, 0)),
        ],
        out_specs=pl.BlockSpec((_B, _C), lambda i: (0, 0)),
        out_shape=jax.ShapeDtypeStruct((_B, _C), jnp.float32),
        scratch_shapes=[
            pltpu.VMEM((_B, _H), jnp.float32),
            pltpu.VMEM((_B, _H), jnp.float32),
            pltpu.VMEM((_B, _H), jnp.float32),
        ],
    )(h4, bidx3, W1, b1.reshape(1, _H), W2, b2.reshape(1, _H), W3,
      b3.reshape(1, _C))


def kernel(x, edge_index, batch_index, W_rel0, b_rel0, W_root0,
           W_rel1, b_rel1, W_root1, W_rel2, b_rel2, W_root2,
           W_rel3, b_rel3, W_root3, W_rel4, b_rel4, W_root4,
           W1, b1, W2, b2, W3, b3):
    src = edge_index[0]
    dst = edge_index[1]
    src32 = src.reshape(32, 5, 20, _K)
    dst32 = dst.reshape(32, 5, 20, _K)
    src16 = src.reshape(16, 10, 20, _K)
    dst16 = dst.reshape(16, 10, 20, _K)
    bidx3 = batch_index.reshape(_N // _RB, _RB, 1)

    p = _seg_sum_layer0(x, src32, dst32)
    h4 = _dense0(p, x, W_rel0, b_rel0, W_root0)
    for Wr, br, Wo in ((W_rel1, b_rel1, W_root1), (W_rel2, b_rel2, W_root2),
                       (W_rel3, b_rel3, W_root3), (W_rel4, b_rel4, W_root4)):
        agg4 = _seg_sum_chunks(h4, src16, dst16)
        h4 = _dense(agg4, h4, Wr, br, Wo)
    return _pool_mlp(h4, bidx3, W1, b1, W2, b2, W3, b3)


# speedup vs baseline: 7.6310x; 1.0556x over previous
import functools

import jax
import jax.numpy as jnp
from jax import lax
from jax.experimental import pallas as pl
from jax.experimental.pallas import tpu as pltpu
from jax.experimental.pallas import tpu_sc as plsc

_N = 10000
_E = 320000
_B = 128
_C = 10
_H = 512
_ACC_ROWS = 10112
_K = 50
_RB = 400
_NEG = float("-inf")


def _zero_acc(r0, acc, sid):
    z = jnp.zeros((16,), jnp.float32)

    @pl.loop(0, 40)
    def _(r):
        @pl.loop(0, 8)
        def _(cc):
            r0[r, pl.ds(cc * 16, 16)] = z

    base = sid * 632

    @pl.loop(0, 15)
    def _(j):
        pltpu.sync_copy(r0.at[pl.ds(0, 40)],
                        acc.at[pl.ds(base + j * 40, 40)])

    pltpu.sync_copy(r0.at[pl.ds(0, 32)], acc.at[pl.ds(base + 600, 32)])


def _edge_pass(table, sidx_all, didx_all, bufs, sems, acc, nb):
    def gstart(j, p):
        pltpu.async_copy(table.at[sidx_all.at[j]], bufs[p], sems[p])

    def gwait(j, p):
        pltpu.make_async_copy(table.at[sidx_all.at[j]], bufs[p],
                              sems[p]).wait()

    def sstart(j, p):
        pltpu.async_copy(bufs[p], acc.at[didx_all.at[j]], sems[p], add=True)

    def swait(j, p):
        pltpu.make_async_copy(bufs[p], acc.at[didx_all.at[j]],
                              sems[p]).wait()

    gstart(0, 0)
    gstart(1, 1)
    nb4 = nb // 4

    @pl.loop(0, nb4)
    def _(b):
        i0 = b * 4
        for u in range(4):
            q = (u + 2) % 4

            def ring(u=u, q=q):
                ii = i0 + u
                if u < 2:
                    @pl.when(b > 0)
                    def _():
                        swait(ii - 2, q)
                    gstart(ii + 2, q)
                else:
                    swait(ii - 2, q)

                    @pl.when(b < nb4 - 1)
                    def _():
                        gstart(ii + 2, q)
                gwait(ii, u)
                sstart(ii, u)

            ring()

    swait(nb - 2, 2)
    swait(nb - 1, 3)


def _edge_blocks(src_tile, dst_tile, table, sidx2, didx2, bufs, sems, acc,
                 isems, nblk, bb):
    def istart(b, p):
        pltpu.make_async_copy(src_tile.at[b], sidx2.at[p], isems[p]).start()
        pltpu.make_async_copy(dst_tile.at[b], didx2.at[p], isems[p]).start()

    def iwait(b, p):
        pltpu.make_async_copy(src_tile.at[b], sidx2.at[p], isems[p]).wait()
        pltpu.make_async_copy(dst_tile.at[b], didx2.at[p], isems[p]).wait()

    istart(0, 0)

    @pl.loop(0, nblk)
    def _(b):
        for p in (0, 1):
            @pl.when(lax.rem(b, 2) == p)
            def _(p=p):
                iwait(b, p)

                @pl.when(b + 1 < nblk)
                def _():
                    istart(b + 1, 1 - p)

                _edge_pass(table, sidx2.at[p], didx2.at[p], bufs, sems,
                           acc, bb)


def _writeout(acc, out2d, sid):
    rpt = 624
    pltpu.sync_copy(acc.at[pl.ds(sid * rpt, rpt)],
                    out2d.at[pl.ds(sid * rpt, rpt)])

    @pl.when(sid == 15)
    def _():
        pltpu.sync_copy(acc.at[pl.ds(16 * rpt, _N - 16 * rpt)],
                        out2d.at[pl.ds(16 * rpt, _N - 16 * rpt)])


def _seg_sum_layer0(x, src4, dst4):
    nblk, bb = src4.shape[1], src4.shape[2]
    mesh = plsc.VectorSubcoreMesh(core_axis_name="core",
                                  subcore_axis_name="subcore")

    @functools.partial(
        pl.kernel,
        out_type=jax.ShapeDtypeStruct((2, _N, 128), jnp.float32),
        mesh=mesh,
        scratch_types=[
            pltpu.VMEM((2, bb, _K), jnp.int32),
            pltpu.VMEM((2, bb, _K), jnp.int32),
            pltpu.VMEM((_K, 128), jnp.float32),
            pltpu.VMEM((_K, 128), jnp.float32),
            pltpu.VMEM((_K, 128), jnp.float32),
            pltpu.VMEM((_K, 128), jnp.float32),
            pltpu.VMEM_SHARED((_ACC_ROWS, 128), jnp.float32),
            pltpu.SemaphoreType.DMA,
            pltpu.SemaphoreType.DMA,
            pltpu.SemaphoreType.DMA,
            pltpu.SemaphoreType.DMA,
            pltpu.SemaphoreType.DMA,
            pltpu.SemaphoreType.DMA,
        ],
    )
    def k(x_hbm, src_hbm, dst_hbm, out_hbm, sidx2, didx2, r0, r1, r2, r3,
          acc, sem0, sem1, sem2, sem3, isem0, isem1):
        core = lax.axis_index("core")
        sid = lax.axis_index("subcore")
        tid = core * 16 + sid
        _zero_acc(r0, acc, sid)
        plsc.subcore_barrier()
        _edge_blocks(src_hbm.at[tid], dst_hbm.at[tid], x_hbm, sidx2, didx2,
                     (r0, r1, r2, r3), (sem0, sem1, sem2, sem3), acc,
                     (isem0, isem1), nblk, bb)
        plsc.subcore_barrier()
        for cid in (0, 1):
            @pl.when(core == cid)
            def _(cid=cid):
                _writeout(acc, out_hbm.at[cid], sid)

    return k(x, src4, dst4)


def _seg_sum_chunks(h4, src4, dst4):
    nblk, bb = src4.shape[1], src4.shape[2]
    mesh = plsc.VectorSubcoreMesh(core_axis_name="core",
                                  subcore_axis_name="subcore")

    @functools.partial(
        pl.kernel,
        out_type=jax.ShapeDtypeStruct((4, _N, 128), jnp.float32),
        mesh=mesh,
        scratch_types=[
            pltpu.VMEM((2, bb, _K), jnp.int32),
            pltpu.VMEM((2, bb, _K), jnp.int32),
            pltpu.VMEM((_K, 128), jnp.float32),
            pltpu.VMEM((_K, 128), jnp.float32),
            pltpu.VMEM((_K, 128), jnp.float32),
            pltpu.VMEM((_K, 128), jnp.float32),
            pltpu.VMEM_SHARED((_ACC_ROWS, 128), jnp.float32),
            pltpu.SemaphoreType.DMA,
            pltpu.SemaphoreType.DMA,
            pltpu.SemaphoreType.DMA,
            pltpu.SemaphoreType.DMA,
            pltpu.SemaphoreType.DMA,
            pltpu.SemaphoreType.DMA,
        ],
    )
    def k(h4_hbm, src_hbm, dst_hbm, out_hbm, sidx2, didx2, r0, r1, r2, r3,
          acc, sem0, sem1, sem2, sem3, isem0, isem1):
        core = lax.axis_index("core")
        sid = lax.axis_index("subcore")
        for cid in (0, 1):
            @pl.when(core == cid)
            def _(cid=cid):
                for j in (0, 1):
                    c = 2 * cid + j
                    _zero_acc(r0, acc, sid)
                    plsc.subcore_barrier()
                    _edge_blocks(src_hbm.at[sid], dst_hbm.at[sid],
                                 h4_hbm.at[c], sidx2, didx2,
                                 (r0, r1, r2, r3), (sem0, sem1, sem2, sem3),
                                 acc, (isem0, isem1), nblk, bb)
                    plsc.subcore_barrier()
                    _writeout(acc, out_hbm.at[c], sid)
                    plsc.subcore_barrier()

    return k(h4, src4, dst4)


def _dense0(p, x, Wrel, brel, Wroot):
    def body(p_ref, x_ref, wr_ref, br_ref, wo_ref, o_ref):
        agg = p_ref[0] + p_ref[1]
        res = jnp.tanh(
            jnp.dot(agg, wr_ref[...], preferred_element_type=jnp.float32)
            + jnp.dot(x_ref[...], wo_ref[...],
                      preferred_element_type=jnp.float32)
            + br_ref[...])
        for c in range(4):
            o_ref[c] = res[:, c * 128:(c + 1) * 128]

    return pl.pallas_call(
        body,
        grid=(_N // _RB,),
        in_specs=[
            pl.BlockSpec((2, _RB, 128), lambda i: (0, i, 0)),
            pl.BlockSpec((_RB, 128), lambda i: (i, 0)),
            pl.BlockSpec((128, _H), lambda i: (0, 0)),
            pl.BlockSpec((1, _H), lambda i: (0, 0)),
            pl.BlockSpec((128, _H), lambda i: (0, 0)),
        ],
        out_specs=pl.BlockSpec((4, _RB, 128), lambda i: (0, i, 0)),
        out_shape=jax.ShapeDtypeStruct((4, _N, 128), jnp.float32),
    )(p, x, Wrel, brel.reshape(1, _H), Wroot)


def _dense(agg4, h4, Wrel, brel, Wroot):
    def body(a_ref, h_ref, wr_ref, br_ref, wo_ref, o_ref):
        agg = jnp.concatenate([a_ref[c] for c in range(4)], axis=1)
        h = jnp.concatenate([h_ref[c] for c in range(4)], axis=1)
        res = jnp.tanh(
            jnp.dot(agg, wr_ref[...], preferred_element_type=jnp.float32)
            + jnp.dot(h, wo_ref[...], preferred_element_type=jnp.float32)
            + br_ref[...])
        for c in range(4):
            o_ref[c] = res[:, c * 128:(c + 1) * 128]

    return pl.pallas_call(
        body,
        grid=(_N // _RB,),
        in_specs=[
            pl.BlockSpec((4, _RB, 128), lambda i: (0, i, 0)),
            pl.BlockSpec((4, _RB, 128), lambda i: (0, i, 0)),
            pl.BlockSpec((_H, _H), lambda i: (0, 0)),
            pl.BlockSpec((1, _H), lambda i: (0, 0)),
            pl.BlockSpec((_H, _H), lambda i: (0, 0)),
        ],
        out_specs=pl.BlockSpec((4, _RB, 128), lambda i: (0, i, 0)),
        out_shape=jax.ShapeDtypeStruct((4, _N, 128), jnp.float32),
    )(agg4, h4, Wrel, brel.reshape(1, _H), Wroot)


def _pool_mlp(h4, bidx3, W1, b1, W2, b2, W3, b3):
    G = _N // _RB

    def body(h_ref, ids_ref, w1_ref, b1_ref, w2_ref, b2_ref, w3_ref, b3_ref,
             o_ref, sum_acc, cnt_acc, max_acc):
        i = pl.program_id(0)

        @pl.when(i == 0)
        def _():
            sum_acc[...] = jnp.zeros((_B, _H), jnp.float32)
            cnt_acc[...] = jnp.zeros((_B, _H), jnp.float32)
            max_acc[...] = jnp.full((_B, _H), _NEG, jnp.float32)

        hb = jnp.concatenate([h_ref[c] for c in range(4)], axis=1)
        ids = ids_ref[0]
        iota_b = lax.broadcasted_iota(jnp.int32, (_RB, _B), 1)
        oh = (ids == iota_b).astype(jnp.float32)
        dn = (((0,), (0,)), ((), ()))
        sum_acc[...] += lax.dot_general(oh, hb, dn,
                                        preferred_element_type=jnp.float32)
        cnt_acc[...] += lax.dot_general(oh, jnp.ones((_RB, _H), jnp.float32),
                                        dn, preferred_element_type=jnp.float32)

        first = ids_ref[0, 0, 0]
        last = ids_ref[0, _RB - 1, 0]
        seg_iota = lax.broadcasted_iota(jnp.int32, (_B, 1), 0)

        def upd(b, _):
            mask = ids == b
            m = jnp.max(jnp.where(mask, hb, _NEG), axis=0,
                        keepdims=True)
            sel = seg_iota == b
            max_acc[...] = jnp.maximum(max_acc[...],
                                       jnp.where(sel, m, _NEG))
            return 0

        lax.fori_loop(first, last + 1, upd, 0)

        @pl.when(i == G - 1)
        def _():
            cnt = cnt_acc[...]
            mean_p = sum_acc[...] / jnp.maximum(cnt, 1.0)
            max_p = jnp.where(cnt > 0.0, max_acc[...], 0.0)
            g = jnp.concatenate([max_p, mean_p], axis=1)
            g = jnp.tanh(jnp.dot(g, w1_ref[...],
                                 preferred_element_type=jnp.float32)
                         + b1_ref[...])
            g = jnp.tanh(jnp.dot(g, w2_ref[...],
                                 preferred_element_type=jnp.float32)
                         + b2_ref[...])
            logits = jnp.dot(g, w3_ref[...],
                             preferred_element_type=jnp.float32) + b3_ref[...]
            mx = jnp.max(logits, axis=1, keepdims=True)
            sh = logits - mx
            lse = jnp.log(jnp.sum(jnp.exp(sh), axis=1, keepdims=True))
            o_ref[...] = sh - lse

    return pl.pallas_call(
        body,
        grid=(G,),
        in_specs=[
            pl.BlockSpec((4, _RB, 128), lambda i: (0, i, 0)),
            pl.BlockSpec((1, _RB, 1), lambda i: (i, 0, 0)),
            pl.BlockSpec((2 * _H, _H), lambda i: (0, 0)),
            pl.BlockSpec((1, _H), lambda i: (0, 0)),
            pl.BlockSpec((_H, _H), lambda i: (0, 0)),
            pl.BlockSpec((1, _H), lambda i: (0, 0)),
            pl.BlockSpec((_H, _C), lambda i: (0, 0)),
            pl.BlockSpec((1, _C), lambda i: (0, 0)),
        ],
        out_specs=pl.BlockSpec((_B, _C), lambda i: (0, 0)),
        out_shape=jax.ShapeDtypeStruct((_B, _C), jnp.float32),
        scratch_shapes=[
            pltpu.VMEM((_B, _H), jnp.float32),
            pltpu.VMEM((_B, _H), jnp.float32),
            pltpu.VMEM((_B, _H), jnp.float32),
        ],
    )(h4, bidx3, W1, b1.reshape(1, _H), W2, b2.reshape(1, _H), W3,
      b3.reshape(1, _C))


def kernel(x, edge_index, batch_index, W_rel0, b_rel0, W_root0,
           W_rel1, b_rel1, W_root1, W_rel2, b_rel2, W_root2,
           W_rel3, b_rel3, W_root3, W_rel4, b_rel4, W_root4,
           W1, b1, W2, b2, W3, b3):
    src = edge_index[0]
    dst = edge_index[1]
    src32 = src.reshape(32, 5, 40, _K)
    dst32 = dst.reshape(32, 5, 40, _K)
    src16 = src.reshape(16, 10, 40, _K)
    dst16 = dst.reshape(16, 10, 40, _K)
    bidx3 = batch_index.reshape(_N // _RB, _RB, 1)

    p = _seg_sum_layer0(x, src32, dst32)
    h4 = _dense0(p, x, W_rel0, b_rel0, W_root0)
    for Wr, br, Wo in ((W_rel1, b_rel1, W_root1), (W_rel2, b_rel2, W_root2),
                       (W_rel3, b_rel3, W_root3), (W_rel4, b_rel4, W_root4)):
        agg4 = _seg_sum_chunks(h4, src16, dst16)
        h4 = _dense(agg4, h4, Wr, br, Wo)
    return _pool_mlp(h4, bidx3, W1, b1, W2, b2, W3, b3)


# speedup vs baseline: 7.6755x; 1.0058x over previous
import functools

import jax
import jax.numpy as jnp
from jax import lax
from jax.experimental import pallas as pl
from jax.experimental.pallas import tpu as pltpu
from jax.experimental.pallas import tpu_sc as plsc

_N = 10000
_E = 320000
_B = 128
_C = 10
_H = 512
_ACC_ROWS = 10112
_K = 50
_RB = 400
_NEG = float("-inf")


def _zero_acc(r0, acc, sid, zsem):
    z = jnp.zeros((16,), jnp.float32)

    @pl.loop(0, 40)
    def _(r):
        @pl.loop(0, 8)
        def _(cc):
            r0[r, pl.ds(cc * 16, 16)] = z

    base = sid * 632

    @pl.loop(0, 15)
    def _(j):
        pltpu.async_copy(r0.at[pl.ds(0, 40)],
                         acc.at[pl.ds(base + j * 40, 40)], zsem)

    pltpu.async_copy(r0.at[pl.ds(0, 32)], acc.at[pl.ds(base + 600, 32)],
                     zsem)

    @pl.loop(0, 15)
    def _(j):
        pltpu.make_async_copy(r0.at[pl.ds(0, 40)],
                              acc.at[pl.ds(base + j * 40, 40)], zsem).wait()

    pltpu.make_async_copy(r0.at[pl.ds(0, 32)],
                          acc.at[pl.ds(base + 600, 32)], zsem).wait()


def _flush(acc, out2d, r0, sid, zsem, do_zero):
    base = sid * 632
    pltpu.sync_copy(acc.at[pl.ds(base, 520)], out2d.at[pl.ds(base, 520)])

    @pl.when(sid < 15)
    def _():
        pltpu.sync_copy(acc.at[pl.ds(base + 520, 112)],
                        out2d.at[pl.ds(base + 520, 112)])

    if do_zero:
        _zero_acc(r0, acc, sid, zsem)


def _edge_pass(table, sidx_all, didx_all, bufs, sems, acc, nb):
    def gstart(j, p):
        pltpu.async_copy(table.at[sidx_all.at[j]], bufs[p], sems[p])

    def gwait(j, p):
        pltpu.make_async_copy(table.at[sidx_all.at[j]], bufs[p],
                              sems[p]).wait()

    def sstart(j, p):
        pltpu.async_copy(bufs[p], acc.at[didx_all.at[j]], sems[p], add=True)

    def swait(j, p):
        pltpu.make_async_copy(bufs[p], acc.at[didx_all.at[j]],
                              sems[p]).wait()

    gstart(0, 0)
    gstart(1, 1)
    nb4 = nb // 4

    @pl.loop(0, nb4)
    def _(b):
        i0 = b * 4
        for u in range(4):
            q = (u + 2) % 4

            def ring(u=u, q=q):
                ii = i0 + u
                if u < 2:
                    @pl.when(b > 0)
                    def _():
                        swait(ii - 2, q)
                    gstart(ii + 2, q)
                else:
                    swait(ii - 2, q)

                    @pl.when(b < nb4 - 1)
                    def _():
                        gstart(ii + 2, q)
                gwait(ii, u)
                sstart(ii, u)

            ring()

    swait(nb - 2, 2)
    swait(nb - 1, 3)


def _edge_blocks(src_tile, dst_tile, table, sidx2, didx2, bufs, sems, acc,
                 isems, nblk, bb):
    def istart(b, p):
        pltpu.make_async_copy(src_tile.at[b], sidx2.at[p], isems[p]).start()
        pltpu.make_async_copy(dst_tile.at[b], didx2.at[p], isems[p]).start()

    def iwait(b, p):
        pltpu.make_async_copy(src_tile.at[b], sidx2.at[p], isems[p]).wait()
        pltpu.make_async_copy(dst_tile.at[b], didx2.at[p], isems[p]).wait()

    istart(0, 0)

    @pl.loop(0, nblk)
    def _(b):
        for p in (0, 1):
            @pl.when(lax.rem(b, 2) == p)
            def _(p=p):
                iwait(b, p)

                @pl.when(b + 1 < nblk)
                def _():
                    istart(b + 1, 1 - p)

                _edge_pass(table, sidx2.at[p], didx2.at[p], bufs, sems,
                           acc, bb)


def _seg_sum_layer0(x, src4, dst4):
    nblk, bb = src4.shape[1], src4.shape[2]
    mesh = plsc.VectorSubcoreMesh(core_axis_name="core",
                                  subcore_axis_name="subcore")

    @functools.partial(
        pl.kernel,
        out_type=jax.ShapeDtypeStruct((2, _N, 128), jnp.float32),
        mesh=mesh,
        scratch_types=[
            pltpu.VMEM((2, bb, _K), jnp.int32),
            pltpu.VMEM((2, bb, _K), jnp.int32),
            pltpu.VMEM((_K, 128), jnp.float32),
            pltpu.VMEM((_K, 128), jnp.float32),
            pltpu.VMEM((_K, 128), jnp.float32),
            pltpu.VMEM((_K, 128), jnp.float32),
            pltpu.VMEM_SHARED((_ACC_ROWS, 128), jnp.float32),
            pltpu.SemaphoreType.DMA,
            pltpu.SemaphoreType.DMA,
            pltpu.SemaphoreType.DMA,
            pltpu.SemaphoreType.DMA,
            pltpu.SemaphoreType.DMA,
            pltpu.SemaphoreType.DMA,
            pltpu.SemaphoreType.DMA,
        ],
    )
    def k(x_hbm, src_hbm, dst_hbm, out_hbm, sidx2, didx2, r0, r1, r2, r3,
          acc, sem0, sem1, sem2, sem3, isem0, isem1, zsem):
        core = lax.axis_index("core")
        sid = lax.axis_index("subcore")
        tid = core * 16 + sid
        _zero_acc(r0, acc, sid, zsem)
        plsc.subcore_barrier()
        _edge_blocks(src_hbm.at[tid], dst_hbm.at[tid], x_hbm, sidx2, didx2,
                     (r0, r1, r2, r3), (sem0, sem1, sem2, sem3), acc,
                     (isem0, isem1), nblk, bb)
        plsc.subcore_barrier()
        for cid in (0, 1):
            @pl.when(core == cid)
            def _(cid=cid):
                _flush(acc, out_hbm.at[cid], r0, sid, zsem, False)

    return k(x, src4, dst4)


def _seg_sum_chunks(h4, src4, dst4):
    nblk, bb = src4.shape[1], src4.shape[2]
    mesh = plsc.VectorSubcoreMesh(core_axis_name="core",
                                  subcore_axis_name="subcore")

    @functools.partial(
        pl.kernel,
        out_type=jax.ShapeDtypeStruct((4, _N, 128), jnp.float32),
        mesh=mesh,
        scratch_types=[
            pltpu.VMEM((2, bb, _K), jnp.int32),
            pltpu.VMEM((2, bb, _K), jnp.int32),
            pltpu.VMEM((_K, 128), jnp.float32),
            pltpu.VMEM((_K, 128), jnp.float32),
            pltpu.VMEM((_K, 128), jnp.float32),
            pltpu.VMEM((_K, 128), jnp.float32),
            pltpu.VMEM_SHARED((_ACC_ROWS, 128), jnp.float32),
            pltpu.SemaphoreType.DMA,
            pltpu.SemaphoreType.DMA,
            pltpu.SemaphoreType.DMA,
            pltpu.SemaphoreType.DMA,
            pltpu.SemaphoreType.DMA,
            pltpu.SemaphoreType.DMA,
            pltpu.SemaphoreType.DMA,
        ],
    )
    def k(h4_hbm, src_hbm, dst_hbm, out_hbm, sidx2, didx2, r0, r1, r2, r3,
          acc, sem0, sem1, sem2, sem3, isem0, isem1, zsem):
        core = lax.axis_index("core")
        sid = lax.axis_index("subcore")
        _zero_acc(r0, acc, sid, zsem)
        for cid in (0, 1):
            @pl.when(core == cid)
            def _(cid=cid):
                for j in (0, 1):
                    c = 2 * cid + j
                    plsc.subcore_barrier()
                    _edge_blocks(src_hbm.at[sid], dst_hbm.at[sid],
                                 h4_hbm.at[c], sidx2, didx2,
                                 (r0, r1, r2, r3), (sem0, sem1, sem2, sem3),
                                 acc, (isem0, isem1), nblk, bb)
                    plsc.subcore_barrier()
                    _flush(acc, out_hbm.at[c], r0, sid, zsem, j == 0)

    return k(h4, src4, dst4)


def _dense0(p, x, Wrel, brel, Wroot):
    def body(p_ref, x_ref, wr_ref, br_ref, wo_ref, o_ref):
        agg = p_ref[0] + p_ref[1]
        res = jnp.tanh(
            jnp.dot(agg, wr_ref[...], preferred_element_type=jnp.float32)
            + jnp.dot(x_ref[...], wo_ref[...],
                      preferred_element_type=jnp.float32)
            + br_ref[...])
        for c in range(4):
            o_ref[c] = res[:, c * 128:(c + 1) * 128]

    return pl.pallas_call(
        body,
        grid=(_N // _RB,),
        in_specs=[
            pl.BlockSpec((2, _RB, 128), lambda i: (0, i, 0)),
            pl.BlockSpec((_RB, 128), lambda i: (i, 0)),
            pl.BlockSpec((128, _H), lambda i: (0, 0)),
            pl.BlockSpec((1, _H), lambda i: (0, 0)),
            pl.BlockSpec((128, _H), lambda i: (0, 0)),
        ],
        out_specs=pl.BlockSpec((4, _RB, 128), lambda i: (0, i, 0)),
        out_shape=jax.ShapeDtypeStruct((4, _N, 128), jnp.float32),
    )(p, x, Wrel, brel.reshape(1, _H), Wroot)


def _dense(agg4, h4, Wrel, brel, Wroot):
    def body(a_ref, h_ref, wr_ref, br_ref, wo_ref, o_ref):
        agg = jnp.concatenate([a_ref[c] for c in range(4)], axis=1)
        h = jnp.concatenate([h_ref[c] for c in range(4)], axis=1)
        res = jnp.tanh(
            jnp.dot(agg, wr_ref[...], preferred_element_type=jnp.float32)
            + jnp.dot(h, wo_ref[...], preferred_element_type=jnp.float32)
            + br_ref[...])
        for c in range(4):
            o_ref[c] = res[:, c * 128:(c + 1) * 128]

    return pl.pallas_call(
        body,
        grid=(_N // _RB,),
        in_specs=[
            pl.BlockSpec((4, _RB, 128), lambda i: (0, i, 0)),
            pl.BlockSpec((4, _RB, 128), lambda i: (0, i, 0)),
            pl.BlockSpec((_H, _H), lambda i: (0, 0)),
            pl.BlockSpec((1, _H), lambda i: (0, 0)),
            pl.BlockSpec((_H, _H), lambda i: (0, 0)),
        ],
        out_specs=pl.BlockSpec((4, _RB, 128), lambda i: (0, i, 0)),
        out_shape=jax.ShapeDtypeStruct((4, _N, 128), jnp.float32),
    )(agg4, h4, Wrel, brel.reshape(1, _H), Wroot)


def _pool_mlp(h4, bidx3, W1, b1, W2, b2, W3, b3):
    G = _N // _RB

    def body(h_ref, ids_ref, w1_ref, b1_ref, w2_ref, b2_ref, w3_ref, b3_ref,
             o_ref, sum_acc, cnt_acc, max_acc):
        i = pl.program_id(0)

        @pl.when(i == 0)
        def _():
            sum_acc[...] = jnp.zeros((_B, _H), jnp.float32)
            cnt_acc[...] = jnp.zeros((_B, _H), jnp.float32)
            max_acc[...] = jnp.full((_B, _H), _NEG, jnp.float32)

        hb = jnp.concatenate([h_ref[c] for c in range(4)], axis=1)
        ids = ids_ref[0]
        iota_b = lax.broadcasted_iota(jnp.int32, (_RB, _B), 1)
        oh = (ids == iota_b).astype(jnp.float32)
        dn = (((0,), (0,)), ((), ()))
        sum_acc[...] += lax.dot_general(oh, hb, dn,
                                        preferred_element_type=jnp.float32)
        cnt_acc[...] += lax.dot_general(oh, jnp.ones((_RB, _H), jnp.float32),
                                        dn, preferred_element_type=jnp.float32)

        first = ids_ref[0, 0, 0]
        last = ids_ref[0, _RB - 1, 0]
        seg_iota = lax.broadcasted_iota(jnp.int32, (_B, 1), 0)

        def upd(b, _):
            mask = ids == b
            m = jnp.max(jnp.where(mask, hb, _NEG), axis=0,
                        keepdims=True)
            sel = seg_iota == b
            max_acc[...] = jnp.maximum(max_acc[...],
                                       jnp.where(sel, m, _NEG))
            return 0

        lax.fori_loop(first, last + 1, upd, 0)

        @pl.when(i == G - 1)
        def _():
            cnt = cnt_acc[...]
            mean_p = sum_acc[...] / jnp.maximum(cnt, 1.0)
            max_p = jnp.where(cnt > 0.0, max_acc[...], 0.0)
            g = jnp.concatenate([max_p, mean_p], axis=1)
            g = jnp.tanh(jnp.dot(g, w1_ref[...],
                                 preferred_element_type=jnp.float32)
                         + b1_ref[...])
            g = jnp.tanh(jnp.dot(g, w2_ref[...],
                                 preferred_element_type=jnp.float32)
                         + b2_ref[...])
            logits = jnp.dot(g, w3_ref[...],
                             preferred_element_type=jnp.float32) + b3_ref[...]
            mx = jnp.max(logits, axis=1, keepdims=True)
            sh = logits - mx
            lse = jnp.log(jnp.sum(jnp.exp(sh), axis=1, keepdims=True))
            o_ref[...] = sh - lse

    return pl.pallas_call(
        body,
        grid=(G,),
        in_specs=[
            pl.BlockSpec((4, _RB, 128), lambda i: (0, i, 0)),
            pl.BlockSpec((1, _RB, 1), lambda i: (i, 0, 0)),
            pl.BlockSpec((2 * _H, _H), lambda i: (0, 0)),
            pl.BlockSpec((1, _H), lambda i: (0, 0)),
            pl.BlockSpec((_H, _H), lambda i: (0, 0)),
            pl.BlockSpec((1, _H), lambda i: (0, 0)),
            pl.BlockSpec((_H, _C), lambda i: (0, 0)),
            pl.BlockSpec((1, _C), lambda i: (0, 0)),
        ],
        out_specs=pl.BlockSpec((_B, _C), lambda i: (0, 0)),
        out_shape=jax.ShapeDtypeStruct((_B, _C), jnp.float32),
        scratch_shapes=[
            pltpu.VMEM((_B, _H), jnp.float32),
            pltpu.VMEM((_B, _H), jnp.float32),
            pltpu.VMEM((_B, _H), jnp.float32),
        ],
    )(h4, bidx3, W1, b1.reshape(1, _H), W2, b2.reshape(1, _H), W3,
      b3.reshape(1, _C))


def kernel(x, edge_index, batch_index, W_rel0, b_rel0, W_root0,
           W_rel1, b_rel1, W_root1, W_rel2, b_rel2, W_root2,
           W_rel3, b_rel3, W_root3, W_rel4, b_rel4, W_root4,
           W1, b1, W2, b2, W3, b3):
    src = edge_index[0]
    dst = edge_index[1]
    src32 = src.reshape(32, 5, 40, _K)
    dst32 = dst.reshape(32, 5, 40, _K)
    src16 = src.reshape(16, 10, 40, _K)
    dst16 = dst.reshape(16, 10, 40, _K)
    bidx3 = batch_index.reshape(_N // _RB, _RB, 1)

    p = _seg_sum_layer0(x, src32, dst32)
    h4 = _dense0(p, x, W_rel0, b_rel0, W_root0)
    for Wr, br, Wo in ((W_rel1, b_rel1, W_root1), (W_rel2, b_rel2, W_root2),
                       (W_rel3, b_rel3, W_root3), (W_rel4, b_rel4, W_root4)):
        agg4 = _seg_sum_chunks(h4, src16, dst16)
        h4 = _dense(agg4, h4, Wr, br, Wo)
    return _pool_mlp(h4, bidx3, W1, b1, W2, b2, W3, b3)


# speedup vs baseline: 7.7641x; 1.0115x over previous
import functools

import jax
import jax.numpy as jnp
from jax import lax
from jax.experimental import pallas as pl
from jax.experimental.pallas import tpu as pltpu
from jax.experimental.pallas import tpu_sc as plsc

_N = 10000
_E = 320000
_B = 128
_C = 10
_H = 512
_ACC_ROWS = 10112
_K = 50
_RB = 400
_NEG = float("-inf")


def _zero_acc(r0, acc, sid, zsem):
    z = jnp.zeros((16,), jnp.float32)

    @pl.loop(0, 40)
    def _(r):
        @pl.loop(0, 8)
        def _(cc):
            r0[r, pl.ds(cc * 16, 16)] = z

    base = sid * 632

    @pl.loop(0, 15)
    def _(j):
        pltpu.async_copy(r0.at[pl.ds(0, 40)],
                         acc.at[pl.ds(base + j * 40, 40)], zsem)

    pltpu.async_copy(r0.at[pl.ds(0, 32)], acc.at[pl.ds(base + 600, 32)],
                     zsem)

    @pl.loop(0, 15)
    def _(j):
        pltpu.make_async_copy(r0.at[pl.ds(0, 40)],
                              acc.at[pl.ds(base + j * 40, 40)], zsem).wait()

    pltpu.make_async_copy(r0.at[pl.ds(0, 32)],
                          acc.at[pl.ds(base + 600, 32)], zsem).wait()


def _flush(acc, out2d, r0, sid, zsem, do_zero):
    base = sid * 632
    pltpu.sync_copy(acc.at[pl.ds(base, 520)], out2d.at[pl.ds(base, 520)])

    @pl.when(sid < 15)
    def _():
        pltpu.sync_copy(acc.at[pl.ds(base + 520, 112)],
                        out2d.at[pl.ds(base + 520, 112)])

    if do_zero:
        _zero_acc(r0, acc, sid, zsem)


def _edge_pass(table, sidx_all, didx_all, bufs, sems, acc, nb):
    def gstart(j, p):
        pltpu.async_copy(table.at[sidx_all.at[j]], bufs[p], sems[p])

    def gwait(j, p):
        pltpu.make_async_copy(table.at[sidx_all.at[j]], bufs[p],
                              sems[p]).wait()

    def sstart(j, p):
        pltpu.async_copy(bufs[p], acc.at[didx_all.at[j]], sems[p], add=True)

    def swait(j, p):
        pltpu.make_async_copy(bufs[p], acc.at[didx_all.at[j]],
                              sems[p]).wait()

    gstart(0, 0)
    gstart(1, 1)
    nb4 = nb // 4

    @pl.loop(0, nb4)
    def _(b):
        i0 = b * 4
        for u in range(4):
            q = (u + 2) % 4

            def ring(u=u, q=q):
                ii = i0 + u
                if u < 2:
                    @pl.when(b > 0)
                    def _():
                        swait(ii - 2, q)
                    gstart(ii + 2, q)
                else:
                    swait(ii - 2, q)

                    @pl.when(b < nb4 - 1)
                    def _():
                        gstart(ii + 2, q)
                gwait(ii, u)
                sstart(ii, u)

            ring()

    swait(nb - 2, 2)
    swait(nb - 1, 3)


def _edge_blocks(src_tile, dst_tile, table, sidx2, didx2, bufs, sems, acc,
                 isems, nblk, bb):
    def istart(b, p):
        pltpu.make_async_copy(src_tile.at[b], sidx2.at[p], isems[p]).start()
        pltpu.make_async_copy(dst_tile.at[b], didx2.at[p], isems[p]).start()

    def iwait(b, p):
        pltpu.make_async_copy(src_tile.at[b], sidx2.at[p], isems[p]).wait()
        pltpu.make_async_copy(dst_tile.at[b], didx2.at[p], isems[p]).wait()

    istart(0, 0)

    @pl.loop(0, nblk)
    def _(b):
        for p in (0, 1):
            @pl.when(lax.rem(b, 2) == p)
            def _(p=p):
                iwait(b, p)

                @pl.when(b + 1 < nblk)
                def _():
                    istart(b + 1, 1 - p)

                _edge_pass(table, sidx2.at[p], didx2.at[p], bufs, sems,
                           acc, bb)


def _seg_sum_layer0(x, src4, dst4):
    nblk, bb = src4.shape[1], src4.shape[2]
    mesh = plsc.VectorSubcoreMesh(core_axis_name="core",
                                  subcore_axis_name="subcore")

    @functools.partial(
        pl.kernel,
        out_type=jax.ShapeDtypeStruct((2, _N, 128), jnp.float32),
        mesh=mesh,
        scratch_types=[
            pltpu.VMEM((2, bb, _K), jnp.int32),
            pltpu.VMEM((2, bb, _K), jnp.int32),
            pltpu.VMEM((_K, 128), jnp.float32),
            pltpu.VMEM((_K, 128), jnp.float32),
            pltpu.VMEM((_K, 128), jnp.float32),
            pltpu.VMEM((_K, 128), jnp.float32),
            pltpu.VMEM_SHARED((_ACC_ROWS, 128), jnp.float32),
            pltpu.SemaphoreType.DMA,
            pltpu.SemaphoreType.DMA,
            pltpu.SemaphoreType.DMA,
            pltpu.SemaphoreType.DMA,
            pltpu.SemaphoreType.DMA,
            pltpu.SemaphoreType.DMA,
            pltpu.SemaphoreType.DMA,
        ],
    )
    def k(x_hbm, src_hbm, dst_hbm, out_hbm, sidx2, didx2, r0, r1, r2, r3,
          acc, sem0, sem1, sem2, sem3, isem0, isem1, zsem):
        core = lax.axis_index("core")
        sid = lax.axis_index("subcore")
        tid = core * 16 + sid
        _zero_acc(r0, acc, sid, zsem)
        plsc.subcore_barrier()
        _edge_blocks(src_hbm.at[tid], dst_hbm.at[tid], x_hbm, sidx2, didx2,
                     (r0, r1, r2, r3), (sem0, sem1, sem2, sem3), acc,
                     (isem0, isem1), nblk, bb)
        plsc.subcore_barrier()
        for cid in (0, 1):
            @pl.when(core == cid)
            def _(cid=cid):
                _flush(acc, out_hbm.at[cid], r0, sid, zsem, False)

    return k(x, src4, dst4)


def _seg_sum_chunks(h4, src4, dst4):
    nblk, bb = src4.shape[1], src4.shape[2]
    mesh = plsc.VectorSubcoreMesh(core_axis_name="core",
                                  subcore_axis_name="subcore")

    @functools.partial(
        pl.kernel,
        out_type=jax.ShapeDtypeStruct((4, _N, 128), jnp.float32),
        mesh=mesh,
        scratch_types=[
            pltpu.VMEM((2, bb, _K), jnp.int32),
            pltpu.VMEM((2, bb, _K), jnp.int32),
            pltpu.VMEM((_K, 128), jnp.float32),
            pltpu.VMEM((_K, 128), jnp.float32),
            pltpu.VMEM((_K, 128), jnp.float32),
            pltpu.VMEM((_K, 128), jnp.float32),
            pltpu.VMEM_SHARED((_ACC_ROWS, 128), jnp.float32),
            pltpu.SemaphoreType.DMA,
            pltpu.SemaphoreType.DMA,
            pltpu.SemaphoreType.DMA,
            pltpu.SemaphoreType.DMA,
            pltpu.SemaphoreType.DMA,
            pltpu.SemaphoreType.DMA,
            pltpu.SemaphoreType.DMA,
        ],
    )
    def k(h4_hbm, src_hbm, dst_hbm, out_hbm, sidx2, didx2, r0, r1, r2, r3,
          acc, sem0, sem1, sem2, sem3, isem0, isem1, zsem):
        core = lax.axis_index("core")
        sid = lax.axis_index("subcore")
        _zero_acc(r0, acc, sid, zsem)
        for cid in (0, 1):
            @pl.when(core == cid)
            def _(cid=cid):
                for j in (0, 1):
                    c = 2 * cid + j
                    plsc.subcore_barrier()
                    _edge_blocks(src_hbm.at[sid], dst_hbm.at[sid],
                                 h4_hbm.at[c], sidx2, didx2,
                                 (r0, r1, r2, r3), (sem0, sem1, sem2, sem3),
                                 acc, (isem0, isem1), nblk, bb)
                    plsc.subcore_barrier()
                    _flush(acc, out_hbm.at[c], r0, sid, zsem, j == 0)

    return k(h4, src4, dst4)


def _dense0(p, x, Wrel, brel, Wroot):
    def body(p_ref, x_ref, wr_ref, br_ref, wo_ref, o_ref):
        agg = p_ref[0] + p_ref[1]
        res = jnp.tanh(
            jnp.dot(agg, wr_ref[...], preferred_element_type=jnp.float32)
            + jnp.dot(x_ref[...], wo_ref[...],
                      preferred_element_type=jnp.float32)
            + br_ref[...])
        for c in range(4):
            o_ref[c] = res[:, c * 128:(c + 1) * 128]

    return pl.pallas_call(
        body,
        grid=(_N // _RB,),
        in_specs=[
            pl.BlockSpec((2, _RB, 128), lambda i: (0, i, 0)),
            pl.BlockSpec((_RB, 128), lambda i: (i, 0)),
            pl.BlockSpec((128, _H), lambda i: (0, 0)),
            pl.BlockSpec((1, _H), lambda i: (0, 0)),
            pl.BlockSpec((128, _H), lambda i: (0, 0)),
        ],
        out_specs=pl.BlockSpec((4, _RB, 128), lambda i: (0, i, 0)),
        out_shape=jax.ShapeDtypeStruct((4, _N, 128), jnp.float32),
    )(p, x, Wrel, brel.reshape(1, _H), Wroot)


def _dense(agg4, h4, Wrel, brel, Wroot):
    def body(a_ref, h_ref, wr_ref, br_ref, wo_ref, o_ref):
        agg = jnp.concatenate([a_ref[c] for c in range(4)], axis=1)
        h = jnp.concatenate([h_ref[c] for c in range(4)], axis=1)
        res = jnp.tanh(
            jnp.dot(agg, wr_ref[...], preferred_element_type=jnp.float32)
            + jnp.dot(h, wo_ref[...], preferred_element_type=jnp.float32)
            + br_ref[...])
        for c in range(4):
            o_ref[c] = res[:, c * 128:(c + 1) * 128]

    return pl.pallas_call(
        body,
        grid=(_N // _RB,),
        in_specs=[
            pl.BlockSpec((4, _RB, 128), lambda i: (0, i, 0)),
            pl.BlockSpec((4, _RB, 128), lambda i: (0, i, 0)),
            pl.BlockSpec((_H, _H), lambda i: (0, 0)),
            pl.BlockSpec((1, _H), lambda i: (0, 0)),
            pl.BlockSpec((_H, _H), lambda i: (0, 0)),
        ],
        out_specs=pl.BlockSpec((4, _RB, 128), lambda i: (0, i, 0)),
        out_shape=jax.ShapeDtypeStruct((4, _N, 128), jnp.float32),
    )(agg4, h4, Wrel, brel.reshape(1, _H), Wroot)


def _dense_pool(agg4, h4, Wrel, brel, Wroot, bidx3, W1, b1, W2, b2, W3, b3):
    G = _N // _RB

    def body(a_ref, h_ref, wr_ref, br_ref, wo_ref, ids_ref, w1_ref, b1_ref,
             w2_ref, b2_ref, w3_ref, b3_ref, o_ref, sum_acc, cnt_acc,
             max_acc):
        i = pl.program_id(0)

        @pl.when(i == 0)
        def _():
            sum_acc[...] = jnp.zeros((_B, _H), jnp.float32)
            cnt_acc[...] = jnp.zeros((_B, _H), jnp.float32)
            max_acc[...] = jnp.full((_B, _H), _NEG, jnp.float32)

        agg = jnp.concatenate([a_ref[c] for c in range(4)], axis=1)
        h = jnp.concatenate([h_ref[c] for c in range(4)], axis=1)
        hb = jnp.tanh(
            jnp.dot(agg, wr_ref[...], preferred_element_type=jnp.float32)
            + jnp.dot(h, wo_ref[...], preferred_element_type=jnp.float32)
            + br_ref[...])
        ids = ids_ref[0]
        iota_b = lax.broadcasted_iota(jnp.int32, (_RB, _B), 1)
        oh = (ids == iota_b).astype(jnp.float32)
        dn = (((0,), (0,)), ((), ()))
        sum_acc[...] += lax.dot_general(oh, hb, dn,
                                        preferred_element_type=jnp.float32)
        cnt_acc[...] += lax.dot_general(oh, jnp.ones((_RB, _H), jnp.float32),
                                        dn, preferred_element_type=jnp.float32)

        first = ids_ref[0, 0, 0]
        last = ids_ref[0, _RB - 1, 0]
        seg_iota = lax.broadcasted_iota(jnp.int32, (_B, 1), 0)

        def upd(b, _):
            mask = ids == b
            m = jnp.max(jnp.where(mask, hb, _NEG), axis=0,
                        keepdims=True)
            sel = seg_iota == b
            max_acc[...] = jnp.maximum(max_acc[...],
                                       jnp.where(sel, m, _NEG))
            return 0

        lax.fori_loop(first, last + 1, upd, 0)

        @pl.when(i == G - 1)
        def _():
            cnt = cnt_acc[...]
            mean_p = sum_acc[...] / jnp.maximum(cnt, 1.0)
            max_p = jnp.where(cnt > 0.0, max_acc[...], 0.0)
            g = jnp.concatenate([max_p, mean_p], axis=1)
            g = jnp.tanh(jnp.dot(g, w1_ref[...],
                                 preferred_element_type=jnp.float32)
                         + b1_ref[...])
            g = jnp.tanh(jnp.dot(g, w2_ref[...],
                                 preferred_element_type=jnp.float32)
                         + b2_ref[...])
            logits = jnp.dot(g, w3_ref[...],
                             preferred_element_type=jnp.float32) + b3_ref[...]
            mx = jnp.max(logits, axis=1, keepdims=True)
            sh = logits - mx
            lse = jnp.log(jnp.sum(jnp.exp(sh), axis=1, keepdims=True))
            o_ref[...] = sh - lse

    return pl.pallas_call(
        body,
        grid=(G,),
        in_specs=[
            pl.BlockSpec((4, _RB, 128), lambda i: (0, i, 0)),
            pl.BlockSpec((4, _RB, 128), lambda i: (0, i, 0)),
            pl.BlockSpec((_H, _H), lambda i: (0, 0)),
            pl.BlockSpec((1, _H), lambda i: (0, 0)),
            pl.BlockSpec((_H, _H), lambda i: (0, 0)),
            pl.BlockSpec((1, _RB, 1), lambda i: (i, 0, 0)),
            pl.BlockSpec((2 * _H, _H), lambda i: (0, 0)),
            pl.BlockSpec((1, _H), lambda i: (0, 0)),
            pl.BlockSpec((_H, _H), lambda i: (0, 0)),
            pl.BlockSpec((1, _H), lambda i: (0, 0)),
            pl.BlockSpec((_H, _C), lambda i: (0, 0)),
            pl.BlockSpec((1, _C), lambda i: (0, 0)),
        ],
        out_specs=pl.BlockSpec((_B, _C), lambda i: (0, 0)),
        out_shape=jax.ShapeDtypeStruct((_B, _C), jnp.float32),
        scratch_shapes=[
            pltpu.VMEM((_B, _H), jnp.float32),
            pltpu.VMEM((_B, _H), jnp.float32),
            pltpu.VMEM((_B, _H), jnp.float32),
        ],
    )(agg4, h4, Wrel, brel.reshape(1, _H), Wroot, bidx3, W1,
      b1.reshape(1, _H), W2, b2.reshape(1, _H), W3, b3.reshape(1, _C))


def kernel(x, edge_index, batch_index, W_rel0, b_rel0, W_root0,
           W_rel1, b_rel1, W_root1, W_rel2, b_rel2, W_root2,
           W_rel3, b_rel3, W_root3, W_rel4, b_rel4, W_root4,
           W1, b1, W2, b2, W3, b3):
    src = edge_index[0]
    dst = edge_index[1]
    src32 = src.reshape(32, 5, 40, _K)
    dst32 = dst.reshape(32, 5, 40, _K)
    src16 = src.reshape(16, 10, 40, _K)
    dst16 = dst.reshape(16, 10, 40, _K)
    bidx3 = batch_index.reshape(_N // _RB, _RB, 1)

    p = _seg_sum_layer0(x, src32, dst32)
    h4 = _dense0(p, x, W_rel0, b_rel0, W_root0)
    for Wr, br, Wo in ((W_rel1, b_rel1, W_root1), (W_rel2, b_rel2, W_root2),
                       (W_rel3, b_rel3, W_root3)):
        agg4 = _seg_sum_chunks(h4, src16, dst16)
        h4 = _dense(agg4, h4, Wr, br, Wo)
    agg4 = _seg_sum_chunks(h4, src16, dst16)
    return _dense_pool(agg4, h4, W_rel4, b_rel4, W_root4, bidx3,
                       W1, b1, W2, b2, W3, b3)
